# baseline (speedup 1.0000x reference)
import jax
import jax.numpy as jnp
from jax import lax
from jax.experimental import pallas as pl
from jax.experimental.pallas import tpu as pltpu
from jax.experimental.pallas import tpu_sc as plsc

_BATCH = 64
_SENT = 512
_HID = 768

_N = _BATCH * _SENT
_NW = 32
_PER_W = _N // _NW
_SPW = _SENT // _NW
_W = _SPW
_S = 4
_CHUNKS = _PER_W // _W
_LANES = 16
_HSL = _HID // _LANES


def _emb_kernel(seqg_hbm, segf_hbm, tok_tab, psgw_hbm, out_hbm,
                idx_v, segf_v, psgw_v, *bufs_and_sems):
    accs = bufs_and_sems[0:_S]
    sgs = bufs_and_sems[_S:2 * _S]
    sos = bufs_and_sems[2 * _S:3 * _S]

    wid = lax.axis_index("s") * 2 + lax.axis_index("c")
    base = wid * _PER_W
    sbase = wid * _SPW

    pltpu.sync_copy(seqg_hbm.at[pl.ds(base, _PER_W)], idx_v)
    pltpu.sync_copy(segf_hbm.at[pl.ds(base * _LANES, _PER_W * _LANES)],
                    segf_v)
    pltpu.sync_copy(psgw_hbm.at[wid], psgw_v)

    def gather(b, coff):
        return pltpu.make_async_copy(
            tok_tab.at[idx_v.at[pl.ds(coff, _W)]], accs[b], sgs[b])

    def out_copy(b, chunk):
        return pltpu.make_async_copy(
            accs[b], out_hbm.at[pl.ds(chunk * _SENT + sbase, _W)], sos[b])

    for b in range(_S):
        gather(b, b * _W).start()

    def chunk_body(chunk, b, first):
        coff = chunk * _W
        gather(b, coff).wait()

        def add_row(j, _):
            sf = segf_v[pl.ds((coff + j) * _LANES, _LANES)]
            for h in range(_HSL):
                sl = pl.ds(h * _LANES, _LANES)
                plsc.addupdate(accs[b].at[j, sl],
                               psgw_v[j, sl] + sf * psgw_v[_W, sl])
            return 0

        lax.fori_loop(0, _W, add_row, 0, unroll=2)

        out_copy(b, chunk).start()

        pb = (b - 1) % _S

        def drain_and_refill():
            out_copy(pb, chunk - 1).wait()

            @pl.when(chunk - 1 + _S < _CHUNKS)
            def _():
                gather(pb, coff + (_S - 1) * _W).start()

        if first:
            @pl.when(chunk >= 1)
            def _():
                drain_and_refill()
        else:
            drain_and_refill()

    def step(it, _):
        for b in range(_S):
            chunk_body(_S * it + b, b, first=(b == 0))
        return 0

    lax.fori_loop(0, _CHUNKS // _S, step, 0)

    out_copy((_CHUNKS - 1) % _S, _CHUNKS - 1).wait()


@jax.jit
def _emb(seqg, segf, token_table, psgw):
    mesh = plsc.VectorSubcoreMesh(core_axis_name="c", subcore_axis_name="s")
    scratch = (
        [pltpu.VMEM((_PER_W,), jnp.int32)]
        + [pltpu.VMEM((_PER_W * _LANES,), jnp.float32)]
        + [pltpu.VMEM((_W + 1, _HID), jnp.float32)]
        + [pltpu.VMEM((_W, _HID), jnp.float32)] * _S
        + [pltpu.SemaphoreType.DMA] * (2 * _S)
    )
    kfn = pl.kernel(
        _emb_kernel,
        out_type=jax.ShapeDtypeStruct((_N, _HID), jnp.float32),
        mesh=mesh,
        scratch_types=scratch,
    )
    return kfn(seqg, segf, token_table, psgw)


def kernel(seq, seg, token_table, position_table, segment_table):
    seqg = (seq.reshape(_BATCH, _NW, _SPW).transpose(1, 0, 2)
            .reshape(-1).astype(jnp.int32))
    segg = (seg.reshape(_BATCH, _NW, _SPW).transpose(1, 0, 2)
            .reshape(-1).astype(jnp.float32))
    segf = jnp.broadcast_to(segg[:, None], (_N, _LANES)).reshape(-1)
    fused = (position_table + segment_table[0][None, :]
             ).reshape(_NW, _SPW, _HID)
    diff = jnp.broadcast_to(
        (segment_table[1] - segment_table[0])[None, None, :],
        (_NW, 1, _HID))
    psgw = jnp.concatenate([fused, diff], axis=1)
    out = _emb(seqg, segf, token_table, psgw)
    return out.reshape(_BATCH, _SENT, _HID)


# speedup vs baseline: 2.4681x; 2.4681x over previous
import jax
import jax.numpy as jnp
from jax import lax
from jax.experimental import pallas as pl
from jax.experimental.pallas import tpu as pltpu
from jax.experimental.pallas import tpu_sc as plsc

_BATCH = 64
_SENT = 512
_HID = 768

_N = _BATCH * _SENT
_NW = 32
_PER_W = _N // _NW
_SPW = _SENT // _NW
_W = _SPW
_S = 4
_CHUNKS = _PER_W // _W
_LANES = 16
_HSL = _HID // _LANES


def _emb_kernel(seqg_hbm, segf_hbm, tok_tab, psgw_hbm, out_hbm,
                idx_v, segf_v, psgw_v, *bufs_and_sems):
    accs = bufs_and_sems[0:_S]
    sgs = bufs_and_sems[_S:2 * _S]
    sos = bufs_and_sems[2 * _S:3 * _S]

    wid = lax.axis_index("s") * 2 + lax.axis_index("c")
    base = wid * _PER_W
    sbase = wid * _SPW

    pltpu.sync_copy(seqg_hbm.at[pl.ds(base, _PER_W)], idx_v)
    pltpu.sync_copy(segf_hbm.at[pl.ds(base * _LANES, _PER_W * _LANES)],
                    segf_v)
    pltpu.sync_copy(psgw_hbm.at[wid], psgw_v)

    def gather(b, coff):
        return pltpu.make_async_copy(
            tok_tab.at[idx_v.at[pl.ds(coff, _W)]], accs[b], sgs[b])

    def out_copy(b, chunk):
        return pltpu.make_async_copy(
            accs[b], out_hbm.at[pl.ds(chunk * _SENT + sbase, _W)], sos[b])

    for b in range(_S):
        gather(b, b * _W).start()

    def chunk_body(chunk, b, first):
        coff = chunk * _W
        gather(b, coff).wait()

        for hg in range(_HSL // 12):
            dr = [psgw_v[_W, pl.ds((hg * 12 + k) * _LANES, _LANES)]
                  for k in range(12)]

            def add_row(j, _, hg=hg, dr=dr):
                sf = segf_v[pl.ds((coff + j) * _LANES, _LANES)]
                for k in range(12):
                    sl = pl.ds((hg * 12 + k) * _LANES, _LANES)
                    plsc.addupdate(accs[b].at[j, sl],
                                   psgw_v[j, sl] + sf * dr[k])
                return 0

            lax.fori_loop(0, _W, add_row, 0, unroll=2)

        out_copy(b, chunk).start()

        pb = (b - 1) % _S

        def drain_and_refill():
            out_copy(pb, chunk - 1).wait()

            @pl.when(chunk - 1 + _S < _CHUNKS)
            def _():
                gather(pb, coff + (_S - 1) * _W).start()

        if first:
            @pl.when(chunk >= 1)
            def _():
                drain_and_refill()
        else:
            drain_and_refill()

    def step(it, _):
        for b in range(_S):
            chunk_body(_S * it + b, b, first=(b == 0))
        return 0

    lax.fori_loop(0, _CHUNKS // _S, step, 0)

    out_copy((_CHUNKS - 1) % _S, _CHUNKS - 1).wait()


@jax.jit
def _emb(seqg, segf, token_table, psgw):
    mesh = plsc.VectorSubcoreMesh(core_axis_name="c", subcore_axis_name="s")
    scratch = (
        [pltpu.VMEM((_PER_W,), jnp.int32)]
        + [pltpu.VMEM((_PER_W * _LANES,), jnp.float32)]
        + [pltpu.VMEM((_W + 1, _HID), jnp.float32)]
        + [pltpu.VMEM((_W, _HID), jnp.float32)] * _S
        + [pltpu.SemaphoreType.DMA] * (2 * _S)
    )
    kfn = pl.kernel(
        _emb_kernel,
        out_type=jax.ShapeDtypeStruct((_N, _HID), jnp.float32),
        mesh=mesh,
        scratch_types=scratch,
    )
    return kfn(seqg, segf, token_table, psgw)


def kernel(seq, seg, token_table, position_table, segment_table):
    seqg = (seq.reshape(_BATCH, _NW, _SPW).transpose(1, 0, 2)
            .reshape(-1).astype(jnp.int32))
    segg = (seg.reshape(_BATCH, _NW, _SPW).transpose(1, 0, 2)
            .reshape(-1).astype(jnp.float32))
    segf = jnp.broadcast_to(segg[:, None], (_N, _LANES)).reshape(-1)
    fused = (position_table + segment_table[0][None, :]
             ).reshape(_NW, _SPW, _HID)
    diff = jnp.broadcast_to(
        (segment_table[1] - segment_table[0])[None, None, :],
        (_NW, 1, _HID))
    psgw = jnp.concatenate([fused, diff], axis=1)
    out = _emb(seqg, segf, token_table, psgw)
    return out.reshape(_BATCH, _SENT, _HID)
